# baseline (speedup 1.0000x reference)
import jax
import jax.numpy as jnp
from jax import lax
from jax.experimental import pallas as pl
from jax.experimental.pallas import tpu as pltpu
from jax.experimental.pallas import tpu_sc as plsc

N = 100000
E = 3200000
HD = 16
OUT = 2

NC, NS = 2, 16
NW = NC * NS
IDX = 128
BLK_E = 640
BLOCKS = E // BLK_E
BPT = 158
N_ACC = 100096
STRIPE = N_ACC // NS

PD = 128 // HD
NP = N // PD
NP_ACC = N_ACC * HD // 128

_mesh = plsc.VectorSubcoreMesh(core_axis_name="c", subcore_axis_name="s")


def _segsum_body(h_hbm, edges_hbm, z2_hbm, out_hbm,
                 idx_v, rows_v, acc, isem, gsem, ssem):
    c = lax.axis_index("c")
    s = lax.axis_index("s")
    w = c * NS + s
    pltpu.sync_copy(z2_hbm, acc.at[pl.ds(s * STRIPE, STRIPE)])
    plsc.subcore_barrier()
    b0 = w * BPT

    def pair(p, carry):
        eba = (b0 + 2 * p) * BLK_E
        ebb = eba + BLK_E
        ia0 = pltpu.async_copy(edges_hbm.at[0, pl.ds(eba, BLK_E)],
                               idx_v.at[0, 0], isem)
        ia1 = pltpu.async_copy(edges_hbm.at[1, pl.ds(eba, BLK_E)],
                               idx_v.at[0, 1], isem)
        ib0 = pltpu.async_copy(edges_hbm.at[0, pl.ds(ebb, BLK_E)],
                               idx_v.at[1, 0], isem)
        ib1 = pltpu.async_copy(edges_hbm.at[1, pl.ds(ebb, BLK_E)],
                               idx_v.at[1, 1], isem)
        ia0.wait()
        ga = pltpu.async_copy(h_hbm.at[idx_v.at[0, 0]], rows_v.at[0], gsem)
        ia1.wait()
        ib0.wait()
        ib1.wait()
        ga.wait()
        sa = pltpu.async_copy(rows_v.at[0], acc.at[idx_v.at[0, 1]], ssem,
                              add=True)
        gb = pltpu.async_copy(h_hbm.at[idx_v.at[1, 0]], rows_v.at[1], gsem)
        gb.wait()
        sb = pltpu.async_copy(rows_v.at[1], acc.at[idx_v.at[1, 1]], ssem,
                              add=True)
        sa.wait()
        sb.wait()
        return carry

    lax.fori_loop(0, jnp.minimum(BPT, BLOCKS - b0) // 2, pair, 0)
    plsc.subcore_barrier()
    pltpu.sync_copy(acc.at[pl.ds(s * STRIPE, STRIPE)],
                    out_hbm.at[pl.ds(c * N_ACC + s * STRIPE, STRIPE)])


def _count_body(edges_hbm, z2_hbm, one_hbm, cnt_hbm, dst_v, ones_v, acc,
                isem, ssem):
    c = lax.axis_index("c")
    s = lax.axis_index("s")
    w = c * NS + s
    pltpu.sync_copy(z2_hbm, acc.at[pl.ds(s * STRIPE, STRIPE)])
    pltpu.sync_copy(one_hbm, ones_v)
    plsc.subcore_barrier()
    b0 = w * BPT

    def pair(p, carry):
        eba = (b0 + 2 * p) * BLK_E
        ebb = eba + BLK_E
        ia = pltpu.async_copy(edges_hbm.at[1, pl.ds(eba, BLK_E)],
                              dst_v.at[0], isem)
        ib = pltpu.async_copy(edges_hbm.at[1, pl.ds(ebb, BLK_E)],
                              dst_v.at[1], isem)
        ia.wait()
        sa = pltpu.async_copy(ones_v, acc.at[dst_v.at[0]], ssem, add=True)
        ib.wait()
        sb = pltpu.async_copy(ones_v, acc.at[dst_v.at[1]], ssem, add=True)
        sa.wait()
        sb.wait()
        return carry

    lax.fori_loop(0, jnp.minimum(BPT, BLOCKS - b0) // 2, pair, 0)
    plsc.subcore_barrier()
    pltpu.sync_copy(acc.at[pl.ds(s * STRIPE, STRIPE)],
                    cnt_hbm.at[pl.ds(c * N_ACC + s * STRIPE, STRIPE)])


_sc_params = pltpu.CompilerParams(use_tc_tiling_on_sc=False)

_segsum = pl.kernel(
    _segsum_body,
    out_type=jax.ShapeDtypeStruct((NC * N_ACC, HD), jnp.float32),
    mesh=_mesh,
    compiler_params=_sc_params,
    scratch_types=[
        pltpu.VMEM((2, 2, BLK_E), jnp.int32),
        pltpu.VMEM((2, BLK_E, HD), jnp.float32),
        pltpu.VMEM_SHARED((N_ACC, HD), jnp.float32),
        pltpu.SemaphoreType.DMA,
        pltpu.SemaphoreType.DMA,
        pltpu.SemaphoreType.DMA,
    ])

_count = pl.kernel(
    _count_body,
    out_type=jax.ShapeDtypeStruct((NC * N_ACC, HD), jnp.float32),
    mesh=_mesh,
    compiler_params=_sc_params,
    scratch_types=[
        pltpu.VMEM((2, BLK_E), jnp.int32),
        pltpu.VMEM((BLK_E, HD), jnp.float32),
        pltpu.VMEM_SHARED((N_ACC, HD), jnp.float32),
        pltpu.SemaphoreType.DMA,
        pltpu.SemaphoreType.DMA,
    ])


def _mean(p0, p1, c0, c1):
    cnt = jnp.maximum(c0[...] + c1[...], 1.0)
    return (p0[...] + p1[...]) / cnt


def _dot(a, b):
    return jnp.dot(a, b, preferred_element_type=jnp.float32)


def _layer1_body(p0, p1, c0, c1, h, wl, bl, wr, out, stats):
    z = _dot(_mean(p0, p1, c0, c1), wl[...]) + _dot(h[...], wr[...]) + bl[...]
    out[...] = z
    rows = lax.broadcasted_iota(jnp.int32, (NP_ACC, 128), 0)
    zm = jnp.where(rows < NP, z, 0.0)
    srow = jnp.sum(zm, axis=0, keepdims=True)
    ssrow = jnp.sum(zm * zm, axis=0, keepdims=True)
    stats[...] = jnp.concatenate(
        [srow, ssrow, jnp.zeros((6, 128), jnp.float32)], axis=0)


def _fold_groups(v):
    acc = v
    for g in range(1, PD):
        acc = acc + jnp.roll(v, g * HD, axis=1)
    return acc


def _bn_relu_body(h, stats, gamma, beta, out):
    st = stats[...]
    mu = _fold_groups(st[0:1]) / float(N)
    ex2 = _fold_groups(st[1:2]) / float(N)
    var = ex2 - mu * mu
    scale = gamma[...] * lax.rsqrt(var + 1e-5)
    out[...] = jnp.maximum((h[...] - mu) * scale + beta[...], 0.0)


def _layer_body(p0, p1, c0, c1, h, wl, bl, wr, out):
    z = _dot(_mean(p0, p1, c0, c1), wl[...]) + _dot(h[...], wr[...]) + bl[...]
    out[...] = jnp.maximum(z, 0.0)


def _layer4_body(p0, p1, c0, c1, h, wl, bl, wr, l1w, l1b, l2w, l2b, out):
    t = _dot(_mean(p0, p1, c0, c1), wl[...]) + _dot(h[...], wr[...]) + bl[...]
    t = jnp.maximum(_dot(t, l1w[...]) + l1b[...], 0.0)
    t = _dot(t, l2w[...]) + l2b[...]
    a = t[:, 0:PD]
    b = t[:, PD:2 * PD]
    m = jnp.maximum(a, b)
    ea = jnp.exp(a - m)
    eb = jnp.exp(b - m)
    s = ea + eb
    out[...] = jnp.concatenate([ea / s, eb / s], axis=1)


def _pspec(c):
    return pl.BlockSpec((None, NP_ACC, 128), lambda i, c=c: (c, 0, 0))


def _wspec(shape):
    return pl.BlockSpec(shape, lambda i: (0,) * len(shape))


def _hspec(d=128):
    return pl.BlockSpec((NP_ACC, d), lambda i: (0, 0))


def _sds(shape):
    return jax.ShapeDtypeStruct(shape, jnp.float32)


_tc_params = pltpu.CompilerParams(vmem_limit_bytes=100 * 1024 * 1024)


def _bd(w):
    return jnp.kron(jnp.eye(PD, dtype=jnp.float32), w)


def kernel(x, edge_index, W1l, b1l, W1r, bn_gamma, bn_beta, W2l, b2l, W2r,
           W3l, b3l, W3r, W4l, b4l, W4r, lin1_W, lin1_b, lin2_W, lin2_b):
    f32 = jnp.float32
    edges = edge_index.astype(jnp.int32)
    z2 = jnp.zeros((STRIPE, HD), f32)
    one = jnp.ones((BLK_E, HD), f32)

    xp = jnp.concatenate(
        [x.reshape(NP, 128), jnp.zeros((NP_ACC - NP, 128), f32)])
    x16 = xp.reshape(N_ACC, HD)

    w1l, w1r = _bd(W1l.T), _bd(W1r.T)
    w2l, w2r = _bd(W2l.T), _bd(W2r.T)
    w3l, w3r = _bd(W3l.T), _bd(W3r.T)
    w4l, w4r = _bd(W4l.T), _bd(W4r.T)
    l1w = _bd(lin1_W.T)
    perm = [2 * g for g in range(PD)] + [2 * g + 1 for g in range(PD)]
    l2w = _bd(lin2_W.T)[:, jnp.array(perm)]
    tile8 = lambda v: jnp.tile(v.reshape(1, -1), (1, PD))
    b1 = tile8(b1l)
    b2 = tile8(b2l)
    b3 = tile8(b3l)
    b4 = tile8(b4l)
    l1b = tile8(lin1_b)
    l2b = jnp.concatenate(
        [jnp.full((1, PD), lin2_b[0], f32), jnp.full((1, PD), lin2_b[1], f32)],
        axis=1)
    gam = tile8(bn_gamma)
    bet = tile8(bn_beta)

    cnt = _count(edges, z2, one).reshape(NC, NP_ACC, 128)
    P1 = _segsum(x16, edges, z2).reshape(NC, NP_ACC, 128)

    layer_specs = [_pspec(0), _pspec(1), _pspec(0), _pspec(1), _hspec(),
                   _wspec((128, 128)), _wspec((1, 128)), _wspec((128, 128))]

    h1pre, stats = pl.pallas_call(
        _layer1_body,
        grid=(1,),
        in_specs=layer_specs,
        out_specs=[_hspec(), pl.BlockSpec((8, 128), lambda i: (0, 0))],
        out_shape=[_sds((NP_ACC, 128)), _sds((8, 128))],
        compiler_params=_tc_params,
    )(P1, P1, cnt, cnt, xp, w1l, b1, w1r)

    h1 = pl.pallas_call(
        _bn_relu_body,
        grid=(1,),
        in_specs=[_hspec(), pl.BlockSpec((8, 128), lambda i: (0, 0)),
                  _wspec((1, 128)), _wspec((1, 128))],
        out_specs=_hspec(),
        out_shape=_sds((NP_ACC, 128)),
        compiler_params=_tc_params,
    )(h1pre, stats, gam, bet)

    def mid_layer(h, wl, bl, wr):
        P = _segsum(h.reshape(N_ACC, HD), edges, z2).reshape(NC, NP_ACC, 128)
        return pl.pallas_call(
            _layer_body,
            grid=(1,),
            in_specs=layer_specs,
            out_specs=_hspec(),
            out_shape=_sds((NP_ACC, 128)),
            compiler_params=_tc_params,
        )(P, P, cnt, cnt, h, wl, bl, wr)

    h2 = mid_layer(h1, w2l, b2, w2r)
    h3 = mid_layer(h2, w3l, b3, w3r)

    P4 = _segsum(h3.reshape(N_ACC, HD), edges, z2).reshape(NC, NP_ACC, 128)
    out_p = pl.pallas_call(
        _layer4_body,
        grid=(1,),
        in_specs=[_pspec(0), _pspec(1), _pspec(0), _pspec(1), _hspec(),
                  _wspec((128, 256)), _wspec((1, 256)), _wspec((128, 256)),
                  _wspec((256, 256)), _wspec((1, 256)),
                  _wspec((256, 16)), _wspec((1, 16))],
        out_specs=_hspec(16),
        out_shape=_sds((NP_ACC, 16)),
        compiler_params=_tc_params,
    )(P4, P4, cnt, cnt, h3, w4l, b4, w4r, l1w, l1b, l2w, l2b)
    return out_p[:NP].reshape(NP, OUT, PD).transpose(0, 2, 1).reshape(N, OUT)


# speedup vs baseline: 1.0890x; 1.0890x over previous
import jax
import jax.numpy as jnp
from jax import lax
from jax.experimental import pallas as pl
from jax.experimental.pallas import tpu as pltpu
from jax.experimental.pallas import tpu_sc as plsc

N = 100000
E = 3200000
HD = 16
OUT = 2

NC, NS = 2, 16
NW = NC * NS
IDX = 128
BLK_E = 800
BLOCKS = E // BLK_E
BPT = 126
N_ACC = 100096
STRIPE = N_ACC // NS

PD = 128 // HD
NP = N // PD
NP_ACC = N_ACC * HD // 128

_mesh = plsc.VectorSubcoreMesh(core_axis_name="c", subcore_axis_name="s")


def _segsum_body(h_hbm, edges_hbm, z2_hbm, out_hbm,
                 idx_v, rows_v, acc, isem, gsem, ssem):
    c = lax.axis_index("c")
    s = lax.axis_index("s")
    w = c * NS + s
    pltpu.sync_copy(z2_hbm, acc.at[pl.ds(s * STRIPE, STRIPE)])
    plsc.subcore_barrier()
    b0 = w * BPT

    def pair(p, carry):
        eba = (b0 + 2 * p) * BLK_E
        ebb = eba + BLK_E
        ia0 = pltpu.async_copy(edges_hbm.at[0, pl.ds(eba, BLK_E)],
                               idx_v.at[0, 0], isem)
        ia1 = pltpu.async_copy(edges_hbm.at[1, pl.ds(eba, BLK_E)],
                               idx_v.at[0, 1], isem)
        ib0 = pltpu.async_copy(edges_hbm.at[0, pl.ds(ebb, BLK_E)],
                               idx_v.at[1, 0], isem)
        ib1 = pltpu.async_copy(edges_hbm.at[1, pl.ds(ebb, BLK_E)],
                               idx_v.at[1, 1], isem)
        ia0.wait()
        ga = pltpu.async_copy(h_hbm.at[idx_v.at[0, 0]], rows_v.at[0], gsem)
        ia1.wait()
        ib0.wait()
        ib1.wait()
        ga.wait()
        sa = pltpu.async_copy(rows_v.at[0], acc.at[idx_v.at[0, 1]], ssem,
                              add=True)
        gb = pltpu.async_copy(h_hbm.at[idx_v.at[1, 0]], rows_v.at[1], gsem)
        gb.wait()
        sb = pltpu.async_copy(rows_v.at[1], acc.at[idx_v.at[1, 1]], ssem,
                              add=True)
        sa.wait()
        sb.wait()
        return carry

    lax.fori_loop(0, jnp.minimum(BPT, BLOCKS - b0) // 2, pair, 0)
    plsc.subcore_barrier()
    pltpu.sync_copy(acc.at[pl.ds(s * STRIPE, STRIPE)],
                    out_hbm.at[pl.ds(c * N_ACC + s * STRIPE, STRIPE)])


def _count_body(edges_hbm, z2_hbm, one_hbm, cnt_hbm, dst_v, ones_v, acc,
                isem, ssem):
    c = lax.axis_index("c")
    s = lax.axis_index("s")
    w = c * NS + s
    pltpu.sync_copy(z2_hbm, acc.at[pl.ds(s * STRIPE, STRIPE)])
    pltpu.sync_copy(one_hbm, ones_v)
    plsc.subcore_barrier()
    b0 = w * BPT

    def pair(p, carry):
        eba = (b0 + 2 * p) * BLK_E
        ebb = eba + BLK_E
        ia = pltpu.async_copy(edges_hbm.at[1, pl.ds(eba, BLK_E)],
                              dst_v.at[0], isem)
        ib = pltpu.async_copy(edges_hbm.at[1, pl.ds(ebb, BLK_E)],
                              dst_v.at[1], isem)
        ia.wait()
        sa = pltpu.async_copy(ones_v, acc.at[dst_v.at[0]], ssem, add=True)
        ib.wait()
        sb = pltpu.async_copy(ones_v, acc.at[dst_v.at[1]], ssem, add=True)
        sa.wait()
        sb.wait()
        return carry

    lax.fori_loop(0, jnp.minimum(BPT, BLOCKS - b0) // 2, pair, 0)
    plsc.subcore_barrier()
    pltpu.sync_copy(acc.at[pl.ds(s * STRIPE, STRIPE)],
                    cnt_hbm.at[pl.ds(c * N_ACC + s * STRIPE, STRIPE)])


_sc_params = pltpu.CompilerParams(use_tc_tiling_on_sc=False)

_segsum = pl.kernel(
    _segsum_body,
    out_type=jax.ShapeDtypeStruct((NC * N_ACC, HD), jnp.float32),
    mesh=_mesh,
    compiler_params=_sc_params,
    scratch_types=[
        pltpu.VMEM((2, 2, BLK_E), jnp.int32),
        pltpu.VMEM((2, BLK_E, HD), jnp.float32),
        pltpu.VMEM_SHARED((N_ACC, HD), jnp.float32),
        pltpu.SemaphoreType.DMA,
        pltpu.SemaphoreType.DMA,
        pltpu.SemaphoreType.DMA,
    ])

_count = pl.kernel(
    _count_body,
    out_type=jax.ShapeDtypeStruct((NC * N_ACC, HD), jnp.float32),
    mesh=_mesh,
    compiler_params=_sc_params,
    scratch_types=[
        pltpu.VMEM((2, BLK_E), jnp.int32),
        pltpu.VMEM((BLK_E, HD), jnp.float32),
        pltpu.VMEM_SHARED((N_ACC, HD), jnp.float32),
        pltpu.SemaphoreType.DMA,
        pltpu.SemaphoreType.DMA,
    ])


def _mean(p0, p1, c0, c1):
    cnt = jnp.maximum(c0[...] + c1[...], 1.0)
    return (p0[...] + p1[...]) / cnt


def _dot(a, b):
    return jnp.dot(a, b, preferred_element_type=jnp.float32)


def _layer1_body(p0, p1, c0, c1, h, wl, bl, wr, out, stats):
    z = _dot(_mean(p0, p1, c0, c1), wl[...]) + _dot(h[...], wr[...]) + bl[...]
    out[...] = z
    rows = lax.broadcasted_iota(jnp.int32, (NP_ACC, 128), 0)
    zm = jnp.where(rows < NP, z, 0.0)
    srow = jnp.sum(zm, axis=0, keepdims=True)
    ssrow = jnp.sum(zm * zm, axis=0, keepdims=True)
    stats[...] = jnp.concatenate(
        [srow, ssrow, jnp.zeros((6, 128), jnp.float32)], axis=0)


def _fold_groups(v):
    acc = v
    for g in range(1, PD):
        acc = acc + jnp.roll(v, g * HD, axis=1)
    return acc


def _bn_relu_body(h, stats, gamma, beta, out):
    st = stats[...]
    mu = _fold_groups(st[0:1]) / float(N)
    ex2 = _fold_groups(st[1:2]) / float(N)
    var = ex2 - mu * mu
    scale = gamma[...] * lax.rsqrt(var + 1e-5)
    out[...] = jnp.maximum((h[...] - mu) * scale + beta[...], 0.0)


def _layer_body(p0, p1, c0, c1, h, wl, bl, wr, out):
    z = _dot(_mean(p0, p1, c0, c1), wl[...]) + _dot(h[...], wr[...]) + bl[...]
    out[...] = jnp.maximum(z, 0.0)


def _layer4_body(p0, p1, c0, c1, h, wl, bl, wr, l1w, l1b, l2w, l2b, out):
    t = _dot(_mean(p0, p1, c0, c1), wl[...]) + _dot(h[...], wr[...]) + bl[...]
    t = jnp.maximum(_dot(t, l1w[...]) + l1b[...], 0.0)
    t = _dot(t, l2w[...]) + l2b[...]
    a = t[:, 0:PD]
    b = t[:, PD:2 * PD]
    m = jnp.maximum(a, b)
    ea = jnp.exp(a - m)
    eb = jnp.exp(b - m)
    s = ea + eb
    out[...] = jnp.concatenate([ea / s, eb / s], axis=1)


def _pspec(c):
    return pl.BlockSpec((None, NP_ACC, 128), lambda i, c=c: (c, 0, 0))


def _wspec(shape):
    return pl.BlockSpec(shape, lambda i: (0,) * len(shape))


def _hspec(d=128):
    return pl.BlockSpec((NP_ACC, d), lambda i: (0, 0))


def _sds(shape):
    return jax.ShapeDtypeStruct(shape, jnp.float32)


_tc_params = pltpu.CompilerParams(vmem_limit_bytes=100 * 1024 * 1024)


def _bd(w):
    return jnp.kron(jnp.eye(PD, dtype=jnp.float32), w)


def kernel(x, edge_index, W1l, b1l, W1r, bn_gamma, bn_beta, W2l, b2l, W2r,
           W3l, b3l, W3r, W4l, b4l, W4r, lin1_W, lin1_b, lin2_W, lin2_b):
    f32 = jnp.float32
    edges = edge_index.astype(jnp.int32)
    z2 = jnp.zeros((STRIPE, HD), f32)
    one = jnp.ones((BLK_E, HD), f32)

    xp = jnp.concatenate(
        [x.reshape(NP, 128), jnp.zeros((NP_ACC - NP, 128), f32)])
    x16 = xp.reshape(N_ACC, HD)

    w1l, w1r = _bd(W1l.T), _bd(W1r.T)
    w2l, w2r = _bd(W2l.T), _bd(W2r.T)
    w3l, w3r = _bd(W3l.T), _bd(W3r.T)
    w4l, w4r = _bd(W4l.T), _bd(W4r.T)
    l1w = _bd(lin1_W.T)
    perm = [2 * g for g in range(PD)] + [2 * g + 1 for g in range(PD)]
    l2w = _bd(lin2_W.T)[:, jnp.array(perm)]
    tile8 = lambda v: jnp.tile(v.reshape(1, -1), (1, PD))
    b1 = tile8(b1l)
    b2 = tile8(b2l)
    b3 = tile8(b3l)
    b4 = tile8(b4l)
    l1b = tile8(lin1_b)
    l2b = jnp.concatenate(
        [jnp.full((1, PD), lin2_b[0], f32), jnp.full((1, PD), lin2_b[1], f32)],
        axis=1)
    gam = tile8(bn_gamma)
    bet = tile8(bn_beta)

    cnt = _count(edges, z2, one).reshape(NC, NP_ACC, 128)
    P1 = _segsum(x16, edges, z2).reshape(NC, NP_ACC, 128)

    layer_specs = [_pspec(0), _pspec(1), _pspec(0), _pspec(1), _hspec(),
                   _wspec((128, 128)), _wspec((1, 128)), _wspec((128, 128))]

    h1pre, stats = pl.pallas_call(
        _layer1_body,
        grid=(1,),
        in_specs=layer_specs,
        out_specs=[_hspec(), pl.BlockSpec((8, 128), lambda i: (0, 0))],
        out_shape=[_sds((NP_ACC, 128)), _sds((8, 128))],
        compiler_params=_tc_params,
    )(P1, P1, cnt, cnt, xp, w1l, b1, w1r)

    h1 = pl.pallas_call(
        _bn_relu_body,
        grid=(1,),
        in_specs=[_hspec(), pl.BlockSpec((8, 128), lambda i: (0, 0)),
                  _wspec((1, 128)), _wspec((1, 128))],
        out_specs=_hspec(),
        out_shape=_sds((NP_ACC, 128)),
        compiler_params=_tc_params,
    )(h1pre, stats, gam, bet)

    def mid_layer(h, wl, bl, wr):
        P = _segsum(h.reshape(N_ACC, HD), edges, z2).reshape(NC, NP_ACC, 128)
        return pl.pallas_call(
            _layer_body,
            grid=(1,),
            in_specs=layer_specs,
            out_specs=_hspec(),
            out_shape=_sds((NP_ACC, 128)),
            compiler_params=_tc_params,
        )(P, P, cnt, cnt, h, wl, bl, wr)

    h2 = mid_layer(h1, w2l, b2, w2r)
    h3 = mid_layer(h2, w3l, b3, w3r)

    P4 = _segsum(h3.reshape(N_ACC, HD), edges, z2).reshape(NC, NP_ACC, 128)
    out_p = pl.pallas_call(
        _layer4_body,
        grid=(1,),
        in_specs=[_pspec(0), _pspec(1), _pspec(0), _pspec(1), _hspec(),
                  _wspec((128, 256)), _wspec((1, 256)), _wspec((128, 256)),
                  _wspec((256, 256)), _wspec((1, 256)),
                  _wspec((256, 16)), _wspec((1, 16))],
        out_specs=_hspec(16),
        out_shape=_sds((NP_ACC, 16)),
        compiler_params=_tc_params,
    )(P4, P4, cnt, cnt, h3, w4l, b4, w4r, l1w, l1b, l2w, l2b)
    return out_p[:NP].reshape(NP, OUT, PD).transpose(0, 2, 1).reshape(N, OUT)
